# baseline (speedup 1.0000x reference)
import functools

import jax
import jax.numpy as jnp
from jax import lax
from jax.experimental import pallas as pl
from jax.experimental.pallas import tpu as pltpu
from jax.experimental.pallas import tpu_sc as plsc

NUM_CORES = 2
NUM_SUBCORES = 16
LANES = 16
GROUP = 256
NUM_WORKERS = NUM_CORES * NUM_SUBCORES
NPARAMS = 17

SC_GROUPS = 3072
TC_BLOCK_GROUPS = 512


def _vq_sc_body(chunk, n_groups, x_hbm, p_hbm, o_hbm, buf, par):
    c = lax.axis_index("c")
    s = lax.axis_index("s")
    wid = s * NUM_CORES + c
    base = wid * chunk

    pltpu.sync_copy(x_hbm.at[pl.ds(base, chunk)], buf)
    pltpu.sync_copy(p_hbm, par)

    a0, a1, a2, a3 = par[0], par[1], par[2], par[3]
    b0, b1, b2, b3 = par[4], par[5], par[6], par[7]
    da1, da2, da3 = par[8], par[9], par[10]
    db1, db2, db3 = par[11], par[12], par[13]
    dc1, dc2, dc3 = par[14], par[15], par[16]
    zero = jnp.zeros((LANES,), jnp.float32)

    def body(g, _):
        goff = g * GROUP
        for u in range(GROUP // (2 * LANES)):
            xo = goff + u * LANES
            yo = xo + (GROUP // 2)
            xv = buf[pl.ds(xo, LANES)]
            yv = buf[pl.ds(yo, LANES)]
            t1 = xv * da1 + yv * db1 + dc1
            t2 = xv * da2 + yv * db2 + dc2
            t3 = xv * da3 + yv * db3 + dc3
            g1 = t1 > zero
            m = jnp.maximum(t1, zero)
            ox = jnp.where(g1, a1, a0)
            oy = jnp.where(g1, b1, b0)
            g2 = t2 > m
            m = jnp.maximum(t2, m)
            ox = jnp.where(g2, a2, ox)
            oy = jnp.where(g2, b2, oy)
            g3 = t3 > m
            ox = jnp.where(g3, a3, ox)
            oy = jnp.where(g3, b3, oy)
            buf[pl.ds(xo, LANES)] = ox
            buf[pl.ds(yo, LANES)] = oy
        return 0

    lax.fori_loop(0, n_groups, body, 0)

    pltpu.sync_copy(buf, o_hbm.at[pl.ds(base, chunk)])


def _vq_tc_body(p_ref, x_ref, o_ref):
    xv = x_ref[:, 0, :]
    yv = x_ref[:, 1, :]
    a0, a1, a2, a3 = p_ref[0], p_ref[1], p_ref[2], p_ref[3]
    b0, b1, b2, b3 = p_ref[4], p_ref[5], p_ref[6], p_ref[7]
    t1 = xv * p_ref[8] + yv * p_ref[11] + p_ref[14]
    t2 = xv * p_ref[9] + yv * p_ref[12] + p_ref[15]
    t3 = xv * p_ref[10] + yv * p_ref[13] + p_ref[16]
    g1 = t1 > 0.0
    m = jnp.maximum(t1, 0.0)
    ox = jnp.where(g1, a1, a0)
    oy = jnp.where(g1, b1, b0)
    g2 = t2 > m
    m = jnp.maximum(t2, m)
    ox = jnp.where(g2, a2, ox)
    oy = jnp.where(g2, b2, oy)
    g3 = t3 > m
    ox = jnp.where(g3, a3, ox)
    oy = jnp.where(g3, b3, oy)
    o_ref[:, 0, :] = ox
    o_ref[:, 1, :] = oy


def _to_points(piece_flat, n_pts):
    p3 = jnp.reshape(piece_flat, (n_pts // 128, 2, 128))
    return jnp.reshape(jnp.transpose(p3, (0, 2, 1)), (n_pts, 2))


def kernel(x, w, Centroid_sum, Centroid_n):
    n, d = x.shape
    total = n * d

    xt = jnp.transpose(jnp.reshape(x, (n // 128, 128, d)), (0, 2, 1))
    xflat = jnp.reshape(xt, (total,))

    a = w[0]
    b = w[1]
    c = -0.5 * jnp.sum(w * w, axis=0)
    scal = jnp.concatenate([
        a, b, a[1:] - a[0], b[1:] - b[0], c[1:] - c[0]
    ]).astype(jnp.float32)
    params = jnp.broadcast_to(scal[:, None], (NPARAMS, LANES))

    sc_words = SC_GROUPS * GROUP
    chunk = sc_words // NUM_WORKERS
    n_groups = chunk // GROUP

    mesh = plsc.VectorSubcoreMesh(
        core_axis_name="c", subcore_axis_name="s",
        num_cores=NUM_CORES, num_subcores=NUM_SUBCORES,
    )
    sc_run = pl.kernel(
        functools.partial(_vq_sc_body, chunk, n_groups),
        out_type=jax.ShapeDtypeStruct((sc_words,), jnp.float32),
        mesh=mesh,
        scratch_types=[
            pltpu.VMEM((chunk,), jnp.float32),
            pltpu.VMEM((NPARAMS, LANES), jnp.float32),
        ],
        compiler_params=pltpu.CompilerParams(needs_layout_passes=False),
    )
    sc_out = sc_run(xflat, params)

    tc_groups = (total - sc_words) // GROUP
    x3d = jnp.reshape(xflat, (total // GROUP, 2, 128))
    tc_out = pl.pallas_call(
        _vq_tc_body,
        grid=(tc_groups // TC_BLOCK_GROUPS,),
        in_specs=[
            pl.BlockSpec(memory_space=pltpu.SMEM),
            pl.BlockSpec(
                (TC_BLOCK_GROUPS, 2, 128),
                lambda i: (SC_GROUPS // TC_BLOCK_GROUPS + i, 0, 0),
            ),
        ],
        out_specs=pl.BlockSpec((TC_BLOCK_GROUPS, 2, 128), lambda i: (i, 0, 0)),
        out_shape=jax.ShapeDtypeStruct((tc_groups, 2, 128), jnp.float32),
    )(scal, x3d)

    sc_pts = _to_points(sc_out, sc_words // 2)
    tc_pts = _to_points(jnp.reshape(tc_out, (total - sc_words,)),
                        (total - sc_words) // 2)
    return jnp.concatenate([sc_pts, tc_pts], axis=0)
